# baseline (speedup 1.0000x reference)
import functools

import jax
import jax.numpy as jnp
from jax import lax
from jax.experimental import pallas as pl
from jax.experimental.pallas import tpu as pltpu
from jax.experimental.pallas import tpu_sc as plsc

NC = 2
NS = 16
NW = NC * NS
LANES = 128
NBUF = 4


def _k_body(T, ks_ref, table_ref, out_ref):
    idx = ks_ref[0, 0, :]
    rows = jax.lax.broadcasted_iota(jnp.int32, (T, idx.shape[0]), 0)
    onehot = jnp.where(rows == idx[None, :], 1.0, 0.0).astype(jnp.float32)
    out_ref[0] = jax.lax.dot_general(
        table_ref[...], onehot,
        dimension_numbers=(((1,), (0,)), ((), ())),
        precision=jax.lax.Precision.HIGHEST,
        preferred_element_type=jnp.float32)


@functools.cache
def _build_k(B, HIST, D, T):
    return pl.pallas_call(
        functools.partial(_k_body, T),
        grid=(HIST,),
        in_specs=[
            pl.BlockSpec((1, 1, B), lambda h: (h, 0, 0)),
            pl.BlockSpec((D, T), lambda h: (0, 0)),
        ],
        out_specs=pl.BlockSpec((1, D, B), lambda h: (h, 0, 0)),
        out_shape=jax.ShapeDtypeStruct((HIST, D, B), jnp.float32),
    )


@functools.cache
def _build_uij(B, D, NU, NJ):
    per_tile = B // NW
    tsu = (NU // LANES - 1) * LANES
    tsj = (NJ // LANES - 1) * LANES
    mesh = plsc.VectorSubcoreMesh(core_axis_name="c", subcore_axis_name="s")

    @functools.partial(
        pl.kernel,
        out_type=(
            jax.ShapeDtypeStruct((D, B), jnp.float32),
            jax.ShapeDtypeStruct((D, B), jnp.float32),
        ),
        mesh=mesh,
        compiler_params=pltpu.CompilerParams(needs_layout_passes=False),
        scratch_types=[
            pltpu.VMEM((per_tile + 16,), jnp.int32),
            pltpu.VMEM((per_tile + 16,), jnp.int32),
        ] + [pltpu.VMEM((D, LANES), jnp.float32)] * (2 * NBUF)
          + [pltpu.VMEM((D, per_tile), jnp.float32)] * 2
          + [pltpu.SemaphoreType.DMA] * (2 * NBUF),
    )
    def sc_kernel(i_hbm, j_hbm, user_t, item_t, tail_u, tail_j,
                  out_i, out_j, sm_i, sm_j, *rest):
        bufs_u = rest[0:NBUF]
        bufs_j = rest[NBUF:2 * NBUF]
        oblk_u, oblk_j = rest[2 * NBUF:2 * NBUF + 2]
        sems_u = rest[2 * NBUF + 2:3 * NBUF + 2]
        sems_j = rest[3 * NBUF + 2:4 * NBUF + 2]
        wid = lax.axis_index("s") * NC + lax.axis_index("c")
        base = pl.multiple_of(wid * per_tile, per_tile)
        row16 = [lax.broadcasted_iota(jnp.int32, (16,), 0) + 16 * r
                 for r in range(D // 16)]

        def make_ops(idx_sm, tab, tail, ts, out_ref, bufs, sems):
            def get(e):
                return idx_sm[pl.ds(e, 16)][0]

            def start(e, b):
                i = get(e)

                @pl.when(i < ts)
                def _():
                    c = pl.multiple_of((i >> 7) * LANES, LANES)
                    pltpu.make_async_copy(
                        tab.at[:, pl.ds(c, LANES)], bufs[b], sems[b]).start()

                @pl.when(i >= ts)
                def _():
                    c = pl.multiple_of(((i - ts) >> 7) * LANES, LANES)
                    pltpu.make_async_copy(
                        tail.at[:, pl.ds(c, LANES)], bufs[b], sems[b]).start()

            def finish(e, b):
                pltpu.make_async_copy(
                    tab.at[:, pl.ds(0, LANES)], bufs[b], sems[b]).wait()
                l16 = jnp.full((16,), get(e) & (LANES - 1), jnp.int32)
                e16 = jnp.full((16,), e, jnp.int32)
                for r in range(D // 16):
                    vals = plsc.load_gather(bufs[b], [row16[r], l16])
                    plsc.store_scatter(out_ref, [row16[r], e16], vals)

            return start, finish

        pltpu.sync_copy(i_hbm.at[pl.ds(base, per_tile)],
                        sm_i.at[pl.ds(0, per_tile)])
        pltpu.sync_copy(j_hbm.at[pl.ds(base, per_tile)],
                        sm_j.at[pl.ds(0, per_tile)])

        start_u, finish_u = make_ops(sm_i, user_t, tail_u, tsu, oblk_u,
                                     bufs_u, sems_u)
        start_j, finish_j = make_ops(sm_j, item_t, tail_j, tsj, oblk_j,
                                     bufs_j, sems_j)

        for b in range(NBUF):
            start_u(b, b)
            start_j(b, b)

        @pl.loop(0, per_tile - NBUF, step=NBUF)
        def _(e):
            for b in range(NBUF):
                finish_u(e + b, b)
                start_u(e + b + NBUF, b)
                finish_j(e + b, b)
                start_j(e + b + NBUF, b)

        for b in range(NBUF):
            finish_u(per_tile - NBUF + b, b)
            finish_j(per_tile - NBUF + b, b)

        pltpu.sync_copy(oblk_u, out_i.at[:, pl.ds(base, per_tile)])
        pltpu.sync_copy(oblk_j, out_j.at[:, pl.ds(base, per_tile)])

    return sc_kernel


def _tail(table_t, n):
    ts = (n // LANES - 1) * LANES
    return jnp.pad(table_t[:, ts:], ((0, 0), (0, 2 * LANES - (n - ts))))


def kernel(i_input, j_input, ks_input, user_embeddings, item_embeddings,
           time_embeddings):
    B, HIST = ks_input.shape
    D = user_embeddings.shape[1]
    T = time_embeddings.shape[0]
    NU = user_embeddings.shape[0]
    NJ = item_embeddings.shape[0]

    user_t = user_embeddings.T
    item_t = item_embeddings.T
    out_i_t, out_j_t = _build_uij(B, D, NU, NJ)(
        i_input.astype(jnp.int32), j_input.astype(jnp.int32),
        user_t, item_t, _tail(user_t, NU), _tail(item_t, NJ))

    ks_t = ks_input.astype(jnp.int32).T.reshape(HIST, 1, B)
    time_t = time_embeddings.T
    out_k_t = _build_k(B, HIST, D, T)(ks_t, time_t)
    out_k = jnp.transpose(out_k_t, (2, 0, 1))
    return (out_i_t.T, out_j_t.T, out_k)
